# baseline (speedup 1.0000x reference)
import functools

import jax
import jax.numpy as jnp
from jax import lax
from jax.experimental import pallas as pl
from jax.experimental.pallas import tpu as pltpu
from jax.experimental.pallas import tpu_sc as plsc

N = 10000
E = 320000
D = 128
H = 128
C = 16

NC = 1
NS = 16
NW = NC * NS
CH = 128
E2 = 327680
ROWS = E2 // CH
RPW = ROWS // NW
IG = 4
RPG = RPW // IG

NG = N
NA = N + 8
RPT = 624
ZR = 208
NDR = 3
ZTAIL = NA - NS * RPT
DTAIL = N - NS * RPT


def _norm_mm_body(x_ref, w1_ref, p1_ref):
    x = x_ref[...]
    mean = jnp.mean(x, axis=0, keepdims=True)
    var = jnp.mean((x - mean) * (x - mean), axis=0, keepdims=True)
    h = (x - mean) * lax.rsqrt(var + 1e-5)
    p1_ref[...] = jnp.dot(h, w1_ref[...], preferred_element_type=jnp.float32)


def _psum(ref):
    if NC == 1:
        return ref[0]
    return ref[0] + ref[1]


def _mid_body(a_ref, d_ref, h2_ref):
    deg = jnp.maximum(_psum(d_ref)[:, 0:1], 1.0)
    h2_ref[...] = jnp.maximum(_psum(a_ref) / deg, 0.0)


def _out_body(a_ref, d_ref, w2_ref, out_ref):
    deg = jnp.maximum(_psum(d_ref)[:, 0:1], 1.0)
    out_ref[...] = jnp.dot(_psum(a_ref) / deg, w2_ref[...],
                           preferred_element_type=jnp.float32)


def _make_agg_body(with_deg):
    def body(*refs):
        if with_deg:
            (tab_hbm, src_hbm, dst_hbm, z128_hbm, ones_hbm,
             acc_out, deg_out, src_i, dst_i, rows0, ones_buf,
             acc_sh, sem0) = refs
        else:
            (tab_hbm, src_hbm, dst_hbm, z128_hbm,
             acc_out, src_i, dst_i, rows0, rows1, acc_sh, sem0, sem1) = refs
        c = lax.axis_index("c")
        s = lax.axis_index("s")

        t0 = s * RPT
        def zero_acc():
            def zcp(j, _):
                o = t0 + j * ZR
                pltpu.sync_copy(z128_hbm.at[pl.ds(o, ZR)],
                                acc_sh.at[pl.ds(o, ZR)])
                return 0
            lax.fori_loop(0, NDR, zcp, 0, unroll=False)

            @pl.when(s == 0)
            def _zero_tail():
                pltpu.sync_copy(z128_hbm.at[pl.ds(NS * RPT, ZTAIL)],
                                acc_sh.at[pl.ds(NS * RPT, ZTAIL)])

        def drain_acc(out_hbm):
            def dcp(j, _):
                o = t0 + j * ZR
                pltpu.sync_copy(acc_sh.at[pl.ds(o, ZR)],
                                out_hbm.at[c, pl.ds(o, ZR)])
                return 0
            lax.fori_loop(0, NDR, dcp, 0, unroll=False)

            @pl.when(s == 0)
            def _drain_tail():
                pltpu.sync_copy(acc_sh.at[pl.ds(NS * RPT, DTAIL)],
                                out_hbm.at[c, pl.ds(NS * RPT, DTAIL)])

        w0 = (c * NS + s) * RPW
        zero_acc()
        if with_deg:
            pltpu.sync_copy(ones_hbm, ones_buf)
            plsc.subcore_barrier()

            def dgroup(g, _):
                base = w0 + g * RPG
                pltpu.sync_copy(dst_hbm.at[pl.ds(base, RPG)], dst_i)

                def dchunk(k, _):
                    pltpu.sync_copy(ones_buf, acc_sh.at[dst_i.at[k]],
                                    add=True)
                    return 0
                lax.fori_loop(0, RPG, dchunk, 0, unroll=False)
                return 0
            lax.fori_loop(0, IG, dgroup, 0, unroll=False)
            plsc.subcore_barrier()
            drain_acc(deg_out)
            zero_acc()
        plsc.subcore_barrier()

        def group(g, _):
            base = w0 + g * RPG
            pltpu.sync_copy(src_hbm.at[pl.ds(base, RPG)], src_i)
            pltpu.sync_copy(dst_hbm.at[pl.ds(base, RPG)], dst_i)

            if with_deg:
                def chunk(k, _):
                    pltpu.async_copy(tab_hbm.at[src_i.at[k]], rows0,
                                     sem0).wait()
                    pltpu.sync_copy(rows0, acc_sh.at[dst_i.at[k]], add=True)
                    return 0
                lax.fori_loop(0, RPG, chunk, 0, unroll=False)
            else:
                def pair(j, _):
                    k0 = 2 * j
                    d0 = pltpu.async_copy(tab_hbm.at[src_i.at[k0]], rows0,
                                          sem0)
                    d1 = pltpu.async_copy(tab_hbm.at[src_i.at[k0 + 1]],
                                          rows1, sem1)
                    d0.wait()
                    pltpu.sync_copy(rows0, acc_sh.at[dst_i.at[k0]], add=True)
                    d1.wait()
                    pltpu.sync_copy(rows1, acc_sh.at[dst_i.at[k0 + 1]],
                                    add=True)
                    return 0
                lax.fori_loop(0, RPG // 2, pair, 0, unroll=False)
            return 0
        lax.fori_loop(0, IG, group, 0, unroll=False)
        plsc.subcore_barrier()
        drain_acc(acc_out)
    return body


_SC_MESH = plsc.VectorSubcoreMesh(
    core_axis_name="c", subcore_axis_name="s", num_cores=NC, num_subcores=NS)

_agg1 = functools.partial(
    pl.kernel,
    out_type=(jax.ShapeDtypeStruct((NC, N, H), jnp.float32),
              jax.ShapeDtypeStruct((NC, N, H), jnp.float32)),
    mesh=_SC_MESH,
    scratch_types=[
        pltpu.VMEM((RPG, CH), jnp.int32),
        pltpu.VMEM((RPG, CH), jnp.int32),
        pltpu.VMEM((CH, H), jnp.float32),
        pltpu.VMEM((CH, H), jnp.float32),
        pltpu.VMEM_SHARED((NA, H), jnp.float32),
        pltpu.SemaphoreType.DMA,
    ],
)(_make_agg_body(True))

_agg2 = functools.partial(
    pl.kernel,
    out_type=jax.ShapeDtypeStruct((NC, N, H), jnp.float32),
    mesh=_SC_MESH,
    scratch_types=[
        pltpu.VMEM((RPG, CH), jnp.int32),
        pltpu.VMEM((RPG, CH), jnp.int32),
        pltpu.VMEM((CH, H), jnp.float32),
        pltpu.VMEM((CH, H), jnp.float32),
        pltpu.VMEM_SHARED((NA, H), jnp.float32),
        pltpu.SemaphoreType.DMA,
        pltpu.SemaphoreType.DMA,
    ],
)(_make_agg_body(False))


def kernel(x, edge_index, W1, W2):
    ei = edge_index.astype(jnp.int32)
    pad = E2 - E
    src = jnp.concatenate([ei[0], jnp.zeros((pad,), jnp.int32)])
    dst = jnp.concatenate([ei[1], jnp.full((pad,), NG, jnp.int32)])
    src2d = src.reshape(ROWS, CH)
    dst2d = dst.reshape(ROWS, CH)
    z128 = jnp.zeros((NA, H), jnp.float32)
    ones128 = jnp.ones((CH, H), jnp.float32)

    p1 = pl.pallas_call(
        _norm_mm_body,
        out_shape=jax.ShapeDtypeStruct((N, H), jnp.float32),
    )(x, W1)

    a1p, degp = _agg1(p1, src2d, dst2d, z128, ones128)

    h2 = pl.pallas_call(
        _mid_body,
        out_shape=jax.ShapeDtypeStruct((N, H), jnp.float32),
    )(a1p, degp)

    a2p = _agg2(h2, src2d, dst2d, z128)

    out = pl.pallas_call(
        _out_body,
        out_shape=jax.ShapeDtypeStruct((N, C), jnp.float32),
    )(a2p, degp, W2)

    return out


# speedup vs baseline: 1.1963x; 1.1963x over previous
import functools

import jax
import jax.numpy as jnp
from jax import lax
from jax.experimental import pallas as pl
from jax.experimental.pallas import tpu as pltpu
from jax.experimental.pallas import tpu_sc as plsc

N = 10000
E = 320000
D = 128
H = 128
C = 16

NC = 2
NS = 16
NW = NC * NS
CH = 128
E2 = 327680
ROWS = E2 // CH
RPW = ROWS // NW
IG = 2
RPG = RPW // IG

NG = N
NA = N + 8
RPT = 624
ZR = 208
NDR = 3
ZTAIL = NA - NS * RPT
DTAIL = N - NS * RPT


def _norm_mm_body(x_ref, w1_ref, p1_ref):
    x = x_ref[...]
    mean = jnp.mean(x, axis=0, keepdims=True)
    var = jnp.mean((x - mean) * (x - mean), axis=0, keepdims=True)
    h = (x - mean) * lax.rsqrt(var + 1e-5)
    p1_ref[...] = jnp.dot(h, w1_ref[...], preferred_element_type=jnp.float32)


def _mid_body(a_ref, d_ref, h2_ref):
    deg = jnp.maximum(d_ref[0][:, 0:1] + d_ref[1][:, 0:1], 1.0)
    h2_ref[...] = jnp.maximum((a_ref[0] + a_ref[1]) / deg, 0.0)


def _out_body(a_ref, d_ref, w2_ref, out_ref):
    deg = jnp.maximum(d_ref[0][:, 0:1] + d_ref[1][:, 0:1], 1.0)
    out_ref[...] = jnp.dot((a_ref[0] + a_ref[1]) / deg, w2_ref[...],
                           preferred_element_type=jnp.float32)


def _make_agg_body(with_deg):
    def body(*refs):
        if with_deg:
            (tab_hbm, src_hbm, dst_hbm, z128_hbm, ones_hbm,
             acc_out, deg_out, src_i, dst_i, rows0, ones_buf,
             acc_sh, sem0) = refs
        else:
            (tab_hbm, src_hbm, dst_hbm, z128_hbm,
             acc_out, src_i, dst_i, rows0, rows1, acc_sh, sem0, sem1) = refs
        c = lax.axis_index("c")
        s = lax.axis_index("s")

        t0 = s * RPT
        def zero_acc():
            def zcp(j, _):
                o = t0 + j * ZR
                pltpu.sync_copy(z128_hbm.at[pl.ds(o, ZR)],
                                acc_sh.at[pl.ds(o, ZR)])
                return 0
            lax.fori_loop(0, NDR, zcp, 0, unroll=False)

            @pl.when(s == 0)
            def _zero_tail():
                pltpu.sync_copy(z128_hbm.at[pl.ds(NS * RPT, ZTAIL)],
                                acc_sh.at[pl.ds(NS * RPT, ZTAIL)])

        def drain_acc(out_hbm):
            def dcp(j, _):
                o = t0 + j * ZR
                pltpu.sync_copy(acc_sh.at[pl.ds(o, ZR)],
                                out_hbm.at[c, pl.ds(o, ZR)])
                return 0
            lax.fori_loop(0, NDR, dcp, 0, unroll=False)

            @pl.when(s == 0)
            def _drain_tail():
                pltpu.sync_copy(acc_sh.at[pl.ds(NS * RPT, DTAIL)],
                                out_hbm.at[c, pl.ds(NS * RPT, DTAIL)])

        w0 = (c * NS + s) * RPW
        zero_acc()
        if with_deg:
            pltpu.sync_copy(ones_hbm, ones_buf)
            plsc.subcore_barrier()

            def dgroup(g, _):
                base = w0 + g * RPG
                pltpu.sync_copy(dst_hbm.at[pl.ds(base, RPG)], dst_i)

                def dchunk(k, _):
                    pltpu.sync_copy(ones_buf, acc_sh.at[dst_i.at[k]],
                                    add=True)
                    return 0
                lax.fori_loop(0, RPG, dchunk, 0, unroll=False)
                return 0
            lax.fori_loop(0, IG, dgroup, 0, unroll=False)
            plsc.subcore_barrier()
            drain_acc(deg_out)
            zero_acc()
        plsc.subcore_barrier()

        def group(g, _):
            base = w0 + g * RPG
            pltpu.sync_copy(src_hbm.at[pl.ds(base, RPG)], src_i)
            pltpu.sync_copy(dst_hbm.at[pl.ds(base, RPG)], dst_i)

            if with_deg:
                def chunk(k, _):
                    pltpu.async_copy(tab_hbm.at[src_i.at[k]], rows0,
                                     sem0).wait()
                    pltpu.sync_copy(rows0, acc_sh.at[dst_i.at[k]], add=True)
                    return 0
                lax.fori_loop(0, RPG, chunk, 0, unroll=False)
            else:
                def pair(j, _):
                    k0 = 2 * j
                    d0 = pltpu.async_copy(tab_hbm.at[src_i.at[k0]], rows0,
                                          sem0)
                    d1 = pltpu.async_copy(tab_hbm.at[src_i.at[k0 + 1]],
                                          rows1, sem1)
                    d0.wait()
                    pltpu.sync_copy(rows0, acc_sh.at[dst_i.at[k0]], add=True)
                    d1.wait()
                    pltpu.sync_copy(rows1, acc_sh.at[dst_i.at[k0 + 1]],
                                    add=True)
                    return 0
                lax.fori_loop(0, RPG // 2, pair, 0, unroll=False)
            return 0
        lax.fori_loop(0, IG, group, 0, unroll=False)
        plsc.subcore_barrier()
        drain_acc(acc_out)
    return body


_SC_MESH = plsc.VectorSubcoreMesh(
    core_axis_name="c", subcore_axis_name="s", num_cores=NC, num_subcores=NS)

_agg1 = functools.partial(
    pl.kernel,
    out_type=(jax.ShapeDtypeStruct((NC, N, H), jnp.float32),
              jax.ShapeDtypeStruct((NC, N, H), jnp.float32)),
    mesh=_SC_MESH,
    scratch_types=[
        pltpu.VMEM((RPG, CH), jnp.int32),
        pltpu.VMEM((RPG, CH), jnp.int32),
        pltpu.VMEM((CH, H), jnp.float32),
        pltpu.VMEM((CH, H), jnp.float32),
        pltpu.VMEM_SHARED((NA, H), jnp.float32),
        pltpu.SemaphoreType.DMA,
    ],
)(_make_agg_body(True))

_agg2 = functools.partial(
    pl.kernel,
    out_type=jax.ShapeDtypeStruct((NC, N, H), jnp.float32),
    mesh=_SC_MESH,
    scratch_types=[
        pltpu.VMEM((RPG, CH), jnp.int32),
        pltpu.VMEM((RPG, CH), jnp.int32),
        pltpu.VMEM((CH, H), jnp.float32),
        pltpu.VMEM((CH, H), jnp.float32),
        pltpu.VMEM_SHARED((NA, H), jnp.float32),
        pltpu.SemaphoreType.DMA,
        pltpu.SemaphoreType.DMA,
    ],
)(_make_agg_body(False))


def kernel(x, edge_index, W1, W2):
    ei = edge_index.astype(jnp.int32)
    pad = E2 - E
    src = jnp.concatenate([ei[0], jnp.zeros((pad,), jnp.int32)])
    dst = jnp.concatenate([ei[1], jnp.full((pad,), NG, jnp.int32)])
    src2d = src.reshape(ROWS, CH)
    dst2d = dst.reshape(ROWS, CH)
    z128 = jnp.zeros((NA, H), jnp.float32)
    ones128 = jnp.ones((CH, H), jnp.float32)

    p1 = pl.pallas_call(
        _norm_mm_body,
        out_shape=jax.ShapeDtypeStruct((N, H), jnp.float32),
    )(x, W1)

    a1p, degp = _agg1(p1, src2d, dst2d, z128, ones128)

    h2 = pl.pallas_call(
        _mid_body,
        out_shape=jax.ShapeDtypeStruct((N, H), jnp.float32),
    )(a1p, degp)

    a2p = _agg2(h2, src2d, dst2d, z128)

    out = pl.pallas_call(
        _out_body,
        out_shape=jax.ShapeDtypeStruct((N, C), jnp.float32),
    )(a2p, degp, W2)

    return out
